# baseline (speedup 1.0000x reference)
import functools

import jax
import jax.numpy as jnp
from jax.experimental import pallas as pl

N = 10000
T = 8
F = 128
E = 320000

SHIFT = 12.0
NB = 1000


def _pre_body(x_ref, w_ref, a2_ref, h_ref, esd_ref):
    xb = x_ref[0]
    h = jnp.dot(xb, w_ref[...], preferred_element_type=jnp.float32)
    h_ref[0] = h
    esd = jnp.dot(h, a2_ref[...], preferred_element_type=jnp.float32)
    esd_ref[0, 0, :, 0] = esd[:, 0]
    esd_ref[0, 1, :, 0] = esd[:, 1]


def _pre(x, W, a_src, a_dst):
    xt = jnp.swapaxes(x, 0, 1)
    a2 = jnp.stack([a_src, a_dst], axis=1)
    grid = (T, N // NB)
    h, esd = pl.pallas_call(
        _pre_body,
        grid=grid,
        in_specs=[
            pl.BlockSpec((1, NB, F), lambda t, i: (t, i, 0)),
            pl.BlockSpec((F, F), lambda t, i: (0, 0)),
            pl.BlockSpec((F, 2), lambda t, i: (0, 0)),
        ],
        out_specs=[
            pl.BlockSpec((1, NB, F), lambda t, i: (t, i, 0)),
            pl.BlockSpec((1, 2, NB, 1), lambda t, i: (t, 0, i, 0)),
        ],
        out_shape=[
            jax.ShapeDtypeStruct((T, N, F), jnp.float32),
            jax.ShapeDtypeStruct((T, 2, N, 1), jnp.float32),
        ],
    )(xt, W, a2)
    return h, esd[:, 0, :, 0], esd[:, 1, :, 0]


def _edges(h, es, ed, src, dst):
    aggs = []
    denoms = []
    for t in range(T):
        e = jax.nn.leaky_relu(es[t, src] + ed[t, dst], negative_slope=0.2)
        w = jnp.exp(e - SHIFT)
        denom = jax.ops.segment_sum(w, dst, num_segments=N)
        agg = jax.ops.segment_sum(w[:, None] * h[t, src, :], dst,
                                  num_segments=N)
        aggs.append(agg)
        denoms.append(denom)
    return jnp.stack(aggs), jnp.stack(denoms)


def _post_body(agg_ref, den_ref, pos_ref, wq_ref, wk_ref, wv_ref, out_ref):
    den = den_ref[:, :, 0]
    agg = agg_ref[...]
    hs = agg / (den[:, :, None] + 1e-30)
    hs = jnp.where(hs > 0, hs, jnp.exp(jnp.minimum(hs, 0.0)) - 1.0)
    z = hs + pos_ref[...][:, None, :]
    zf = z.reshape(T * NB, F)
    q = jnp.dot(zf, wq_ref[...], preferred_element_type=jnp.float32)
    k = jnp.dot(zf, wk_ref[...], preferred_element_type=jnp.float32)
    v = jnp.dot(zf, wv_ref[...], preferred_element_type=jnp.float32)
    q = q.reshape(T, NB, F) * (1.0 / (F ** 0.5))
    k = k.reshape(T, NB, F)
    v = v.reshape(T, NB, F)
    for t in range(T):
        ss = [jnp.sum(q[t] * k[s], axis=-1) for s in range(t + 1)]
        m = ss[0]
        for s in range(1, t + 1):
            m = jnp.maximum(m, ss[s])
        ws = [jnp.exp(s_ - m) for s_ in ss]
        dsum = ws[0]
        for s in range(1, t + 1):
            dsum = dsum + ws[s]
        acc = ws[0][:, None] * v[0]
        for s in range(1, t + 1):
            acc = acc + ws[s][:, None] * v[s]
        out_ref[:, t, :] = acc / dsum[:, None]


def _post(agg, denom, pos_emb, Wq, Wk, Wv):
    den3 = denom.reshape(T, N, 1)
    grid = (N // NB,)
    return pl.pallas_call(
        _post_body,
        grid=grid,
        in_specs=[
            pl.BlockSpec((T, NB, F), lambda i: (0, i, 0)),
            pl.BlockSpec((T, NB, 1), lambda i: (0, i, 0)),
            pl.BlockSpec((T, F), lambda i: (0, 0)),
            pl.BlockSpec((F, F), lambda i: (0, 0)),
            pl.BlockSpec((F, F), lambda i: (0, 0)),
            pl.BlockSpec((F, F), lambda i: (0, 0)),
        ],
        out_specs=pl.BlockSpec((NB, T, F), lambda i: (i, 0, 0)),
        out_shape=jax.ShapeDtypeStruct((N, T, F), jnp.float32),
    )(agg, den3, pos_emb, Wq, Wk, Wv)


def kernel(x, edge_index, W, a_src, a_dst, pos_emb, Wq, Wk, Wv):
    src = edge_index[0]
    dst = edge_index[1]
    h, es, ed = _pre(x, W, a_src, a_dst)
    agg, denom = _edges(h, es, ed, src, dst)
    return _post(agg, denom, pos_emb, Wq, Wk, Wv)


# speedup vs baseline: 23.4596x; 23.4596x over previous
import functools

import jax
import jax.numpy as jnp
from jax import lax
from jax.experimental import pallas as pl
from jax.experimental.pallas import tpu as pltpu
from jax.experimental.pallas import tpu_sc as plsc

N = 10000
T = 8
F = 128
E = 320000

SHIFT = 12.0
NB = 1000


def _pre_body(x_ref, w_ref, a2_ref, hx_ref, esd_ref):
    for t in range(T):
        xb = x_ref[:, t, :]
        h = jnp.dot(xb, w_ref[...], preferred_element_type=jnp.float32)
        esd = jnp.dot(h, a2_ref[...], preferred_element_type=jnp.float32)
        hx_ref[t, :, :128] = h
        hx_ref[t, :, 128:] = jnp.broadcast_to(esd[:, 0:1], (NB, 16))
        esd_ref[t, 0, :, 0] = esd[:, 0]
        esd_ref[t, 1, :, 0] = esd[:, 1]


def _pre(x, W, a_src, a_dst):
    a2 = jnp.stack([a_src, a_dst], axis=1)
    grid = (N // NB,)
    hx, esd = pl.pallas_call(
        _pre_body,
        grid=grid,
        in_specs=[
            pl.BlockSpec((NB, T, F), lambda i: (i, 0, 0)),
            pl.BlockSpec((F, F), lambda i: (0, 0)),
            pl.BlockSpec((F, 2), lambda i: (0, 0)),
        ],
        out_specs=[
            pl.BlockSpec((T, NB, 144), lambda i: (0, i, 0)),
            pl.BlockSpec((T, 2, NB, 1), lambda i: (0, 0, i, 0)),
        ],
        out_shape=[
            jax.ShapeDtypeStruct((T, N, 144), jnp.float32),
            jax.ShapeDtypeStruct((T, 2, N, 1), jnp.float32),
        ],
    )(x, W, a2)
    return hx, esd[:, 1, :, 0]


NTILE = 16
K = 48
NCHUNK = 418
EPTP = K * NCHUNK
EP = NTILE * EPTP
FP = 144
EDW = 16
ZROWS = 40
WTILES = 10
WROWS = N // WTILES
TPC = T // 2


def _edges_sc_body(hx_hbm, ed_hbm, src_hbm, dst_hbm, out_hbm,
                   si0, si1, di0, di1, sci0, sci1, g0, g1, e0, e1, s0, s1,
                   zbuf, accum,
                   sem_i0, sem_i1, sem_g0, sem_g1, sem_e0, sem_e1,
                   sem_s0, sem_s1):
    c = lax.axis_index("c")
    sid = lax.axis_index("s")
    sis = (si0, si1)
    dis = (di0, di1)
    scis = (sci0, sci1)
    gbufs = (g0, g1)
    ebufs = (e0, e1)
    sbufs = (s0, s1)
    sems_i = (sem_i0, sem_i1)
    sems_g = (sem_g0, sem_g1)
    sems_e = (sem_e0, sem_e1)
    sems_s = (sem_s0, sem_s1)
    row0 = sid * WROWS
    my_src = src_hbm.at[sid]
    my_dst = dst_hbm.at[sid]

    zv = jnp.zeros((16,), jnp.float32)

    def z_body(i, carry):
        for v in range(FP // 16):
            zbuf[i, pl.ds(v * 16, 16)] = zv
        return carry

    lax.fori_loop(0, ZROWS, z_body, 0)

    def stage_idx(cidx, b):
        pltpu.async_copy(my_src.at[cidx], sis[b], sems_i[b])
        pltpu.async_copy(my_dst.at[cidx], dis[b], sems_i[b])

    def wait_idx(cidx, b):
        pltpu.make_async_copy(my_src.at[cidx], sis[b], sems_i[b]).wait()
        pltpu.make_async_copy(my_dst.at[cidx], dis[b], sems_i[b]).wait()

    def t_body(tt, carry):
        t = c * TPC + tt
        hx_t = hx_hbm.at[t]
        ed_t = ed_hbm.at[t]

        @pl.when(sid < WTILES)
        def _():
            for i in range(WROWS // ZROWS):
                pltpu.sync_copy(zbuf,
                                accum.at[pl.ds(row0 + i * ZROWS, ZROWS)])

        plsc.subcore_barrier()

        stage_idx(0, 0)
        stage_idx(1, 1)
        wait_idx(0, 0)
        pltpu.async_copy(hx_t.at[si0], g0, sem_g0)
        pltpu.async_copy(ed_t.at[di0], e0, sem_e0)

        def pair_body(p, carry2):
            for b in range(2):
                cidx = p * 2 + b
                nb = 1 - b
                si_b, di_b, sci_b = sis[b], dis[b], scis[b]
                gb, eb, sb = gbufs[b], ebufs[b], sbufs[b]

                @pl.when(cidx + 1 < NCHUNK)
                def _():
                    wait_idx(cidx + 1, nb)
                    pltpu.async_copy(hx_t.at[sis[nb]], gbufs[nb],
                                     sems_g[nb])
                    pltpu.async_copy(ed_t.at[dis[nb]], ebufs[nb],
                                     sems_e[nb])

                @pl.when(cidx >= 2)
                def _():
                    pltpu.make_async_copy(sb, accum.at[sci_b],
                                          sems_s[b]).wait()

                pltpu.make_async_copy(hx_t.at[si_b], gb, sems_g[b]).wait()
                pltpu.make_async_copy(ed_t.at[di_b], eb, sems_e[b]).wait()

                for k in range(K // 16):
                    sl = pl.ds(k * 16, 16)
                    rows = lax.iota(jnp.int32, 16) + (k * 16)
                    dv = di_b[sl]
                    sci_b[sl] = dv
                    es_v = plsc.load_gather(gb, [rows,
                                                 jnp.full((16,), F,
                                                          jnp.int32)])
                    ed_v = plsc.load_gather(eb, [rows,
                                                 jnp.zeros((16,),
                                                           jnp.int32)])
                    e = es_v + ed_v
                    e = jnp.where(e > 0, e, e * 0.2)
                    w = jnp.exp(e - SHIFT)
                    gid = sid * EPTP + cidx * K + rows
                    w = jnp.where(gid < E, w, 0.0)
                    plsc.store_scatter(sb, [rows,
                                            jnp.full((16,), F, jnp.int32)],
                                       w)

                def edge_body(j, carry3):
                    wv = jnp.full((16,), sb[j, pl.ds(F, 16)][0])
                    for v in range(F // 16):
                        sl = pl.ds(v * 16, 16)
                        sb[j, sl] = gb[j, sl] * wv
                    return carry3

                lax.fori_loop(0, K, edge_body, 0, unroll=2)

                @pl.when(cidx + 2 < NCHUNK)
                def _():
                    stage_idx(cidx + 2, b)

                pltpu.async_copy(sb, accum.at[sci_b], sems_s[b], add=True)
            return carry2

        lax.fori_loop(0, NCHUNK // 2, pair_body, 0)
        for b in range(2):
            pltpu.make_async_copy(sbufs[b], accum.at[scis[b]],
                                  sems_s[b]).wait()
        plsc.subcore_barrier()

        @pl.when(sid < WTILES)
        def _():
            pltpu.sync_copy(accum.at[pl.ds(row0, WROWS)],
                            out_hbm.at[t].at[pl.ds(row0, WROWS)])

        return carry

    lax.fori_loop(0, TPC, t_body, 0)


def _edges(hx, ed, src, dst):
    mesh = plsc.VectorSubcoreMesh(core_axis_name="c", subcore_axis_name="s")
    ed16 = jnp.broadcast_to(ed[:, :, None], (T, N, EDW))
    pad = jnp.zeros((EP - E,), jnp.int32)
    srcp = jnp.concatenate([src, pad]).reshape(NTILE, NCHUNK, K)
    dstp = jnp.concatenate([dst, pad]).reshape(NTILE, NCHUNK, K)
    out144 = pl.kernel(
        _edges_sc_body,
        out_type=jax.ShapeDtypeStruct((T, N, FP), jnp.float32),
        mesh=mesh,
        compiler_params=pltpu.CompilerParams(needs_layout_passes=False,
                                             use_tc_tiling_on_sc=False),
        scratch_types=[
            pltpu.VMEM((K,), jnp.int32),
            pltpu.VMEM((K,), jnp.int32),
            pltpu.VMEM((K,), jnp.int32),
            pltpu.VMEM((K,), jnp.int32),
            pltpu.VMEM((K,), jnp.int32),
            pltpu.VMEM((K,), jnp.int32),
            pltpu.VMEM((K, FP), jnp.float32),
            pltpu.VMEM((K, FP), jnp.float32),
            pltpu.VMEM((K, EDW), jnp.float32),
            pltpu.VMEM((K, EDW), jnp.float32),
            pltpu.VMEM((K, FP), jnp.float32),
            pltpu.VMEM((K, FP), jnp.float32),
            pltpu.VMEM((ZROWS, FP), jnp.float32),
            pltpu.VMEM_SHARED((N, FP), jnp.float32),
            pltpu.SemaphoreType.DMA,
            pltpu.SemaphoreType.DMA,
            pltpu.SemaphoreType.DMA,
            pltpu.SemaphoreType.DMA,
            pltpu.SemaphoreType.DMA,
            pltpu.SemaphoreType.DMA,
            pltpu.SemaphoreType.DMA,
            pltpu.SemaphoreType.DMA,
        ],
    )(hx, ed16, srcp, dstp)
    return out144


def _post_body(agg_ref, pos_ref, wq_ref, wk_ref, wv_ref, out_ref):
    blk = agg_ref[...]
    den = blk[:, :, F:F + 1]
    agg = blk[:, :, :F]
    hs = agg / (den + 1e-30)
    hs = jnp.where(hs > 0, hs, jnp.exp(jnp.minimum(hs, 0.0)) - 1.0)
    z = hs + pos_ref[...][:, None, :]
    zf = z.reshape(T * NB, F)
    q = jnp.dot(zf, wq_ref[...], preferred_element_type=jnp.float32)
    k = jnp.dot(zf, wk_ref[...], preferred_element_type=jnp.float32)
    v = jnp.dot(zf, wv_ref[...], preferred_element_type=jnp.float32)
    q = q.reshape(T, NB, F) * (1.0 / (F ** 0.5))
    k = k.reshape(T, NB, F)
    v = v.reshape(T, NB, F)
    for t in range(T):
        ss = [jnp.sum(q[t] * k[s], axis=-1) for s in range(t + 1)]
        m = ss[0]
        for s in range(1, t + 1):
            m = jnp.maximum(m, ss[s])
        ws = [jnp.exp(s_ - m) for s_ in ss]
        dsum = ws[0]
        for s in range(1, t + 1):
            dsum = dsum + ws[s]
        acc = ws[0][:, None] * v[0]
        for s in range(1, t + 1):
            acc = acc + ws[s][:, None] * v[s]
        out_ref[:, t, :] = acc / dsum[:, None]


def _post(out144, pos_emb, Wq, Wk, Wv):
    grid = (N // NB,)
    return pl.pallas_call(
        _post_body,
        grid=grid,
        in_specs=[
            pl.BlockSpec((T, NB, FP), lambda i: (0, i, 0)),
            pl.BlockSpec((T, F), lambda i: (0, 0)),
            pl.BlockSpec((F, F), lambda i: (0, 0)),
            pl.BlockSpec((F, F), lambda i: (0, 0)),
            pl.BlockSpec((F, F), lambda i: (0, 0)),
        ],
        out_specs=pl.BlockSpec((NB, T, F), lambda i: (i, 0, 0)),
        out_shape=jax.ShapeDtypeStruct((N, T, F), jnp.float32),
    )(out144, pos_emb, Wq, Wk, Wv)


def kernel(x, edge_index, W, a_src, a_dst, pos_emb, Wq, Wk, Wv):
    src = edge_index[0]
    dst = edge_index[1]
    hx, ed = _pre(x, W, a_src, a_dst)
    out144 = _edges(hx, ed, src, dst)
    return _post(out144, pos_emb, Wq, Wk, Wv)


# speedup vs baseline: 48.7670x; 2.0788x over previous
import functools

import jax
import jax.numpy as jnp
from jax import lax
from jax.experimental import pallas as pl
from jax.experimental.pallas import tpu as pltpu
from jax.experimental.pallas import tpu_sc as plsc

N = 10000
T = 8
F = 128
E = 320000

SHIFT = 12.0
NB = 1000


def _pre_body(x_ref, w_ref, a2_ref, hx_ref, esd_ref):
    for t in range(T):
        xb = x_ref[:, t, :]
        h = jnp.dot(xb, w_ref[...], preferred_element_type=jnp.float32)
        esd = jnp.dot(h, a2_ref[...], preferred_element_type=jnp.float32)
        hx_ref[t, :, :128] = h
        hx_ref[t, :, 128:] = jnp.broadcast_to(esd[:, 0:1], (NB, 16))
        esd_ref[t, 0, :, 0] = esd[:, 0]
        esd_ref[t, 1, :, 0] = esd[:, 1]


def _pre(x, W, a_src, a_dst):
    a2 = jnp.stack([a_src, a_dst], axis=1)
    grid = (N // NB,)
    hx, esd = pl.pallas_call(
        _pre_body,
        grid=grid,
        in_specs=[
            pl.BlockSpec((NB, T, F), lambda i: (i, 0, 0)),
            pl.BlockSpec((F, F), lambda i: (0, 0)),
            pl.BlockSpec((F, 2), lambda i: (0, 0)),
        ],
        out_specs=[
            pl.BlockSpec((T, NB, 144), lambda i: (0, i, 0)),
            pl.BlockSpec((T, 2, NB, 1), lambda i: (0, 0, i, 0)),
        ],
        out_shape=[
            jax.ShapeDtypeStruct((T, N, 144), jnp.float32),
            jax.ShapeDtypeStruct((T, 2, N, 1), jnp.float32),
        ],
    )(x, W, a2)
    return hx, esd[:, 1, :, 0]


NTILE = 16
K = 48
NCHUNK = 418
EPTP = K * NCHUNK
EP = NTILE * EPTP
FP = 144
EDW = 16
ZROWS = 40
WTILES = 10
WROWS = N // WTILES
TPC = T // 2


def _edges_sc_body(hx_hbm, ed_hbm, src_hbm, dst_hbm, out_hbm,
                   si0, si1, di0, di1, sci0, sci1, g0, g1, e0, e1, s0, s1,
                   zbuf, accum,
                   sem_i0, sem_i1, sem_g0, sem_g1, sem_e0, sem_e1,
                   sem_s0, sem_s1):
    c = lax.axis_index("c")
    sid = lax.axis_index("s")
    sis = (si0, si1)
    dis = (di0, di1)
    scis = (sci0, sci1)
    gbufs = (g0, g1)
    ebufs = (e0, e1)
    sbufs = (s0, s1)
    sems_i = (sem_i0, sem_i1)
    sems_g = (sem_g0, sem_g1)
    sems_e = (sem_e0, sem_e1)
    sems_s = (sem_s0, sem_s1)
    row0 = sid * WROWS
    my_src = src_hbm.at[sid]
    my_dst = dst_hbm.at[sid]

    zv = jnp.zeros((16,), jnp.float32)

    def z_body(i, carry):
        for v in range(FP // 16):
            zbuf[i, pl.ds(v * 16, 16)] = zv
        return carry

    lax.fori_loop(0, ZROWS, z_body, 0)

    def stage_idx(cidx, b):
        pltpu.async_copy(my_src.at[cidx], sis[b], sems_i[b])
        pltpu.async_copy(my_dst.at[cidx], dis[b], sems_i[b])

    def wait_idx(cidx, b):
        pltpu.make_async_copy(my_src.at[cidx], sis[b], sems_i[b]).wait()
        pltpu.make_async_copy(my_dst.at[cidx], dis[b], sems_i[b]).wait()

    def t_body(tt, carry):
        t = c * TPC + tt
        hx_t = hx_hbm.at[t]
        ed_t = ed_hbm.at[t]

        @pl.when(sid < WTILES)
        def _():
            for i in range(WROWS // ZROWS):
                pltpu.sync_copy(zbuf,
                                accum.at[pl.ds(row0 + i * ZROWS, ZROWS)])

        plsc.subcore_barrier()

        stage_idx(0, 0)
        stage_idx(1, 1)
        wait_idx(0, 0)
        pltpu.async_copy(hx_t.at[si0], g0, sem_g0)
        pltpu.async_copy(ed_t.at[di0], e0, sem_e0)

        def pair_body(p, carry2):
            for b in range(2):
                cidx = p * 2 + b
                nb = 1 - b
                si_b, di_b, sci_b = sis[b], dis[b], scis[b]
                gb, eb, sb = gbufs[b], ebufs[b], sbufs[b]

                @pl.when(cidx + 1 < NCHUNK)
                def _():
                    wait_idx(cidx + 1, nb)
                    pltpu.async_copy(hx_t.at[sis[nb]], gbufs[nb],
                                     sems_g[nb])
                    pltpu.async_copy(ed_t.at[dis[nb]], ebufs[nb],
                                     sems_e[nb])

                @pl.when(cidx >= 2)
                def _():
                    pltpu.make_async_copy(sb, accum.at[sci_b],
                                          sems_s[b]).wait()

                pltpu.make_async_copy(hx_t.at[si_b], gb, sems_g[b]).wait()
                pltpu.make_async_copy(ed_t.at[di_b], eb, sems_e[b]).wait()

                for k in range(K // 16):
                    sl = pl.ds(k * 16, 16)
                    rows = lax.iota(jnp.int32, 16) + (k * 16)
                    dv = di_b[sl]
                    sci_b[sl] = dv
                    es_v = plsc.load_gather(gb, [rows,
                                                 jnp.full((16,), F,
                                                          jnp.int32)])
                    ed_v = plsc.load_gather(eb, [rows,
                                                 jnp.zeros((16,),
                                                           jnp.int32)])
                    e = es_v + ed_v
                    e = jnp.where(e > 0, e, e * 0.2)
                    w = jnp.exp(e - SHIFT)
                    gid = sid * EPTP + cidx * K + rows
                    w = jnp.where(gid < E, w, 0.0)
                    plsc.store_scatter(sb, [rows,
                                            jnp.full((16,), F, jnp.int32)],
                                       w)

                @plsc.parallel_loop(0, K, 1, unroll=4)
                def _(j):
                    wv = jnp.full((16,), sb[j, pl.ds(F, 16)][0])
                    for v in range(F // 16):
                        sl = pl.ds(v * 16, 16)
                        sb[j, sl] = gb[j, sl] * wv

                @pl.when(cidx + 2 < NCHUNK)
                def _():
                    stage_idx(cidx + 2, b)

                pltpu.async_copy(sb, accum.at[sci_b], sems_s[b], add=True)
            return carry2

        lax.fori_loop(0, NCHUNK // 2, pair_body, 0)
        for b in range(2):
            pltpu.make_async_copy(sbufs[b], accum.at[scis[b]],
                                  sems_s[b]).wait()
        plsc.subcore_barrier()

        @pl.when(sid < WTILES)
        def _():
            pltpu.sync_copy(accum.at[pl.ds(row0, WROWS)],
                            out_hbm.at[t].at[pl.ds(row0, WROWS)])

        return carry

    lax.fori_loop(0, TPC, t_body, 0)


def _edges(hx, ed, src, dst):
    mesh = plsc.VectorSubcoreMesh(core_axis_name="c", subcore_axis_name="s")
    ed16 = jnp.broadcast_to(ed[:, :, None], (T, N, EDW))
    pad = jnp.zeros((EP - E,), jnp.int32)
    srcp = jnp.concatenate([src, pad]).reshape(NTILE, NCHUNK, K)
    dstp = jnp.concatenate([dst, pad]).reshape(NTILE, NCHUNK, K)
    out144 = pl.kernel(
        _edges_sc_body,
        out_type=jax.ShapeDtypeStruct((T, N, FP), jnp.float32),
        mesh=mesh,
        compiler_params=pltpu.CompilerParams(needs_layout_passes=False,
                                             use_tc_tiling_on_sc=False),
        scratch_types=[
            pltpu.VMEM((K,), jnp.int32),
            pltpu.VMEM((K,), jnp.int32),
            pltpu.VMEM((K,), jnp.int32),
            pltpu.VMEM((K,), jnp.int32),
            pltpu.VMEM((K,), jnp.int32),
            pltpu.VMEM((K,), jnp.int32),
            pltpu.VMEM((K, FP), jnp.float32),
            pltpu.VMEM((K, FP), jnp.float32),
            pltpu.VMEM((K, EDW), jnp.float32),
            pltpu.VMEM((K, EDW), jnp.float32),
            pltpu.VMEM((K, FP), jnp.float32),
            pltpu.VMEM((K, FP), jnp.float32),
            pltpu.VMEM((ZROWS, FP), jnp.float32),
            pltpu.VMEM_SHARED((N, FP), jnp.float32),
            pltpu.SemaphoreType.DMA,
            pltpu.SemaphoreType.DMA,
            pltpu.SemaphoreType.DMA,
            pltpu.SemaphoreType.DMA,
            pltpu.SemaphoreType.DMA,
            pltpu.SemaphoreType.DMA,
            pltpu.SemaphoreType.DMA,
            pltpu.SemaphoreType.DMA,
        ],
    )(hx, ed16, srcp, dstp)
    return out144


def _post_body(agg_ref, pos_ref, wq_ref, wk_ref, wv_ref, out_ref):
    blk = agg_ref[...]
    den = blk[:, :, F:F + 1]
    agg = blk[:, :, :F]
    hs = agg / (den + 1e-30)
    hs = jnp.where(hs > 0, hs, jnp.exp(jnp.minimum(hs, 0.0)) - 1.0)
    z = hs + pos_ref[...][:, None, :]
    zf = z.reshape(T * NB, F)
    q = jnp.dot(zf, wq_ref[...], preferred_element_type=jnp.float32)
    k = jnp.dot(zf, wk_ref[...], preferred_element_type=jnp.float32)
    v = jnp.dot(zf, wv_ref[...], preferred_element_type=jnp.float32)
    q = q.reshape(T, NB, F) * (1.0 / (F ** 0.5))
    k = k.reshape(T, NB, F)
    v = v.reshape(T, NB, F)
    for t in range(T):
        ss = [jnp.sum(q[t] * k[s], axis=-1) for s in range(t + 1)]
        m = ss[0]
        for s in range(1, t + 1):
            m = jnp.maximum(m, ss[s])
        ws = [jnp.exp(s_ - m) for s_ in ss]
        dsum = ws[0]
        for s in range(1, t + 1):
            dsum = dsum + ws[s]
        acc = ws[0][:, None] * v[0]
        for s in range(1, t + 1):
            acc = acc + ws[s][:, None] * v[s]
        out_ref[:, t, :] = acc / dsum[:, None]


def _post(out144, pos_emb, Wq, Wk, Wv):
    grid = (N // NB,)
    return pl.pallas_call(
        _post_body,
        grid=grid,
        in_specs=[
            pl.BlockSpec((T, NB, FP), lambda i: (0, i, 0)),
            pl.BlockSpec((T, F), lambda i: (0, 0)),
            pl.BlockSpec((F, F), lambda i: (0, 0)),
            pl.BlockSpec((F, F), lambda i: (0, 0)),
            pl.BlockSpec((F, F), lambda i: (0, 0)),
        ],
        out_specs=pl.BlockSpec((NB, T, F), lambda i: (i, 0, 0)),
        out_shape=jax.ShapeDtypeStruct((N, T, F), jnp.float32),
    )(out144, pos_emb, Wq, Wk, Wv)


def kernel(x, edge_index, W, a_src, a_dst, pos_emb, Wq, Wk, Wv):
    src = edge_index[0]
    dst = edge_index[1]
    hx, ed = _pre(x, W, a_src, a_dst)
    out144 = _edges(hx, ed, src, dst)
    return _post(out144, pos_emb, Wq, Wk, Wv)


# speedup vs baseline: 49.0636x; 1.0061x over previous
import functools

import jax
import jax.numpy as jnp
from jax import lax
from jax.experimental import pallas as pl
from jax.experimental.pallas import tpu as pltpu
from jax.experimental.pallas import tpu_sc as plsc

N = 10000
T = 8
F = 128
E = 320000

SHIFT = 12.0
NB = 1000


def _pre_body(x_ref, w_ref, a2_ref, hx_ref, esd_ref):
    for t in range(T):
        xb = x_ref[:, t, :]
        h = jnp.dot(xb, w_ref[...], preferred_element_type=jnp.float32)
        esd = jnp.dot(h, a2_ref[...], preferred_element_type=jnp.float32)
        hx_ref[t, :, :128] = h
        hx_ref[t, :, 128:] = jnp.broadcast_to(esd[:, 0:1], (NB, 16))
        esd_ref[t, 0, :, 0] = esd[:, 0]
        esd_ref[t, 1, :, 0] = esd[:, 1]


def _pre(x, W, a_src, a_dst):
    a2 = jnp.stack([a_src, a_dst], axis=1)
    grid = (N // NB,)
    hx, esd = pl.pallas_call(
        _pre_body,
        grid=grid,
        in_specs=[
            pl.BlockSpec((NB, T, F), lambda i: (i, 0, 0)),
            pl.BlockSpec((F, F), lambda i: (0, 0)),
            pl.BlockSpec((F, 2), lambda i: (0, 0)),
        ],
        out_specs=[
            pl.BlockSpec((T, NB, 144), lambda i: (0, i, 0)),
            pl.BlockSpec((T, 2, NB, 1), lambda i: (0, 0, i, 0)),
        ],
        out_shape=[
            jax.ShapeDtypeStruct((T, N, 144), jnp.float32),
            jax.ShapeDtypeStruct((T, 2, N, 1), jnp.float32),
        ],
    )(x, W, a2)
    return hx, esd[:, 1, :, 0]


NTILE = 16
K = 48
NCHUNK = 418
EPTP = K * NCHUNK
EP = NTILE * EPTP
FP = 144
EDW = 16
ZROWS = 40
WTILES = 10
WROWS = N // WTILES
TPC = T // 2


def _edges_sc_body(hx_hbm, ed_hbm, src_hbm, dst_hbm, out_hbm,
                   si0, si1, di0, di1, sci0, sci1, g0, g1, e0, e1, s0, s1,
                   zbuf, accum,
                   sem_i0, sem_i1, sem_g0, sem_g1, sem_e0, sem_e1,
                   sem_s0, sem_s1):
    c = lax.axis_index("c")
    sid = lax.axis_index("s")
    sis = (si0, si1)
    dis = (di0, di1)
    scis = (sci0, sci1)
    gbufs = (g0, g1)
    ebufs = (e0, e1)
    sbufs = (s0, s1)
    sems_i = (sem_i0, sem_i1)
    sems_g = (sem_g0, sem_g1)
    sems_e = (sem_e0, sem_e1)
    sems_s = (sem_s0, sem_s1)
    row0 = sid * WROWS
    my_src = src_hbm.at[sid]
    my_dst = dst_hbm.at[sid]

    zv = jnp.zeros((16,), jnp.float32)

    def z_body(i, carry):
        for v in range(FP // 16):
            zbuf[i, pl.ds(v * 16, 16)] = zv
        return carry

    lax.fori_loop(0, ZROWS, z_body, 0)

    def stage_idx(cidx, b):
        pltpu.async_copy(my_src.at[cidx], sis[b], sems_i[b])
        pltpu.async_copy(my_dst.at[cidx], dis[b], sems_i[b])

    def wait_idx(cidx, b):
        pltpu.make_async_copy(my_src.at[cidx], sis[b], sems_i[b]).wait()
        pltpu.make_async_copy(my_dst.at[cidx], dis[b], sems_i[b]).wait()

    def t_body(tt, carry):
        t = c * TPC + tt
        hx_t = hx_hbm.at[t]
        ed_t = ed_hbm.at[t]

        @pl.when(sid < WTILES)
        def _():
            for i in range(WROWS // ZROWS):
                pltpu.sync_copy(zbuf,
                                accum.at[pl.ds(row0 + i * ZROWS, ZROWS)])

        plsc.subcore_barrier()

        stage_idx(0, 0)
        stage_idx(1, 1)
        wait_idx(0, 0)
        pltpu.async_copy(hx_t.at[si0], g0, sem_g0)
        pltpu.async_copy(ed_t.at[di0], e0, sem_e0)

        def pair_body(p, carry2):
            for b in range(2):
                cidx = p * 2 + b
                nb = 1 - b
                si_b, di_b, sci_b = sis[b], dis[b], scis[b]
                gb, eb, sb = gbufs[b], ebufs[b], sbufs[b]

                @pl.when(cidx + 1 < NCHUNK)
                def _():
                    wait_idx(cidx + 1, nb)
                    pltpu.async_copy(hx_t.at[sis[nb]], gbufs[nb],
                                     sems_g[nb])
                    pltpu.async_copy(ed_t.at[dis[nb]], ebufs[nb],
                                     sems_e[nb])

                @pl.when(cidx >= 2)
                def _():
                    pltpu.make_async_copy(sb, accum.at[sci_b],
                                          sems_s[b]).wait()

                pltpu.make_async_copy(hx_t.at[si_b], gb, sems_g[b]).wait()
                pltpu.make_async_copy(ed_t.at[di_b], eb, sems_e[b]).wait()

                for k in range(K // 16):
                    sl = pl.ds(k * 16, 16)
                    rows = lax.iota(jnp.int32, 16) + (k * 16)
                    dv = di_b[sl]
                    sci_b[sl] = dv
                    es_v = plsc.load_gather(gb, [rows,
                                                 jnp.full((16,), F,
                                                          jnp.int32)])
                    ed_v = plsc.load_gather(eb, [rows,
                                                 jnp.zeros((16,),
                                                           jnp.int32)])
                    e = es_v + ed_v
                    e = jnp.where(e > 0, e, e * 0.2)
                    w = jnp.exp(e - SHIFT)
                    gid = sid * EPTP + cidx * K + rows
                    w = jnp.where(gid < E, w, 0.0)
                    plsc.store_scatter(sb, [rows,
                                            jnp.full((16,), F, jnp.int32)],
                                       w)

                @plsc.parallel_loop(0, K, 1, unroll=8)
                def _(j):
                    wv = jnp.full((16,), sb[j, pl.ds(F, 16)][0])
                    for v in range(F // 16):
                        sl = pl.ds(v * 16, 16)
                        sb[j, sl] = gb[j, sl] * wv

                @pl.when(cidx + 2 < NCHUNK)
                def _():
                    stage_idx(cidx + 2, b)

                pltpu.async_copy(sb, accum.at[sci_b], sems_s[b], add=True)
            return carry2

        lax.fori_loop(0, NCHUNK // 2, pair_body, 0)
        for b in range(2):
            pltpu.make_async_copy(sbufs[b], accum.at[scis[b]],
                                  sems_s[b]).wait()
        plsc.subcore_barrier()

        @pl.when(sid < WTILES)
        def _():
            pltpu.sync_copy(accum.at[pl.ds(row0, WROWS)],
                            out_hbm.at[t].at[pl.ds(row0, WROWS)])

        return carry

    lax.fori_loop(0, TPC, t_body, 0)


def _edges(hx, ed, src, dst):
    mesh = plsc.VectorSubcoreMesh(core_axis_name="c", subcore_axis_name="s")
    ed16 = jnp.broadcast_to(ed[:, :, None], (T, N, EDW))
    pad = jnp.zeros((EP - E,), jnp.int32)
    srcp = jnp.concatenate([src, pad]).reshape(NTILE, NCHUNK, K)
    dstp = jnp.concatenate([dst, pad]).reshape(NTILE, NCHUNK, K)
    out144 = pl.kernel(
        _edges_sc_body,
        out_type=jax.ShapeDtypeStruct((T, N, FP), jnp.float32),
        mesh=mesh,
        compiler_params=pltpu.CompilerParams(needs_layout_passes=False,
                                             use_tc_tiling_on_sc=False),
        scratch_types=[
            pltpu.VMEM((K,), jnp.int32),
            pltpu.VMEM((K,), jnp.int32),
            pltpu.VMEM((K,), jnp.int32),
            pltpu.VMEM((K,), jnp.int32),
            pltpu.VMEM((K,), jnp.int32),
            pltpu.VMEM((K,), jnp.int32),
            pltpu.VMEM((K, FP), jnp.float32),
            pltpu.VMEM((K, FP), jnp.float32),
            pltpu.VMEM((K, EDW), jnp.float32),
            pltpu.VMEM((K, EDW), jnp.float32),
            pltpu.VMEM((K, FP), jnp.float32),
            pltpu.VMEM((K, FP), jnp.float32),
            pltpu.VMEM((ZROWS, FP), jnp.float32),
            pltpu.VMEM_SHARED((N, FP), jnp.float32),
            pltpu.SemaphoreType.DMA,
            pltpu.SemaphoreType.DMA,
            pltpu.SemaphoreType.DMA,
            pltpu.SemaphoreType.DMA,
            pltpu.SemaphoreType.DMA,
            pltpu.SemaphoreType.DMA,
            pltpu.SemaphoreType.DMA,
            pltpu.SemaphoreType.DMA,
        ],
    )(hx, ed16, srcp, dstp)
    return out144


def _post_body(agg_ref, pos_ref, wq_ref, wk_ref, wv_ref, out_ref):
    blk = agg_ref[...]
    den = blk[:, :, F:F + 1]
    agg = blk[:, :, :F]
    hs = agg / (den + 1e-30)
    hs = jnp.where(hs > 0, hs, jnp.exp(jnp.minimum(hs, 0.0)) - 1.0)
    z = hs + pos_ref[...][:, None, :]
    zf = z.reshape(T * NB, F)
    q = jnp.dot(zf, wq_ref[...], preferred_element_type=jnp.float32)
    k = jnp.dot(zf, wk_ref[...], preferred_element_type=jnp.float32)
    v = jnp.dot(zf, wv_ref[...], preferred_element_type=jnp.float32)
    q = q.reshape(T, NB, F) * (1.0 / (F ** 0.5))
    k = k.reshape(T, NB, F)
    v = v.reshape(T, NB, F)
    for t in range(T):
        ss = [jnp.sum(q[t] * k[s], axis=-1) for s in range(t + 1)]
        m = ss[0]
        for s in range(1, t + 1):
            m = jnp.maximum(m, ss[s])
        ws = [jnp.exp(s_ - m) for s_ in ss]
        dsum = ws[0]
        for s in range(1, t + 1):
            dsum = dsum + ws[s]
        acc = ws[0][:, None] * v[0]
        for s in range(1, t + 1):
            acc = acc + ws[s][:, None] * v[s]
        out_ref[:, t, :] = acc / dsum[:, None]


def _post(out144, pos_emb, Wq, Wk, Wv):
    grid = (N // NB,)
    return pl.pallas_call(
        _post_body,
        grid=grid,
        in_specs=[
            pl.BlockSpec((T, NB, FP), lambda i: (0, i, 0)),
            pl.BlockSpec((T, F), lambda i: (0, 0)),
            pl.BlockSpec((F, F), lambda i: (0, 0)),
            pl.BlockSpec((F, F), lambda i: (0, 0)),
            pl.BlockSpec((F, F), lambda i: (0, 0)),
        ],
        out_specs=pl.BlockSpec((NB, T, F), lambda i: (i, 0, 0)),
        out_shape=jax.ShapeDtypeStruct((N, T, F), jnp.float32),
    )(out144, pos_emb, Wq, Wk, Wv)


def kernel(x, edge_index, W, a_src, a_dst, pos_emb, Wq, Wk, Wv):
    src = edge_index[0]
    dst = edge_index[1]
    hx, ed = _pre(x, W, a_src, a_dst)
    out144 = _edges(hx, ed, src, dst)
    return _post(out144, pos_emb, Wq, Wk, Wv)


# speedup vs baseline: 50.4410x; 1.0281x over previous
import functools

import jax
import jax.numpy as jnp
from jax import lax
from jax.experimental import pallas as pl
from jax.experimental.pallas import tpu as pltpu
from jax.experimental.pallas import tpu_sc as plsc

N = 10000
T = 8
F = 128
E = 320000

SHIFT = 12.0
NB = 1000


def _pre_body(x_ref, w_ref, a2_ref, hx_ref, esd_ref):
    for t in range(T):
        xb = x_ref[:, t, :]
        h = jnp.dot(xb, w_ref[...], preferred_element_type=jnp.float32)
        esd = jnp.dot(h, a2_ref[...], preferred_element_type=jnp.float32)
        hx_ref[t, :, :128] = h
        hx_ref[t, :, 128:] = jnp.broadcast_to(esd[:, 0:1], (NB, 16))
        esd_ref[t, 0, :, 0] = esd[:, 0]
        esd_ref[t, 1, :, 0] = esd[:, 1]


def _pre(x, W, a_src, a_dst):
    a2 = jnp.stack([a_src, a_dst], axis=1)
    grid = (N // NB,)
    hx, esd = pl.pallas_call(
        _pre_body,
        grid=grid,
        in_specs=[
            pl.BlockSpec((NB, T, F), lambda i: (i, 0, 0)),
            pl.BlockSpec((F, F), lambda i: (0, 0)),
            pl.BlockSpec((F, 2), lambda i: (0, 0)),
        ],
        out_specs=[
            pl.BlockSpec((T, NB, 144), lambda i: (0, i, 0)),
            pl.BlockSpec((T, 2, NB, 1), lambda i: (0, 0, i, 0)),
        ],
        out_shape=[
            jax.ShapeDtypeStruct((T, N, 144), jnp.float32),
            jax.ShapeDtypeStruct((T, 2, N, 1), jnp.float32),
        ],
    )(x, W, a2)
    return hx, esd[:, 1, :, 0]


NTILE = 16
K = 64
NCHUNK = 314
EPTP = K * NCHUNK
EP = NTILE * EPTP
FP = 144
FPA = 136
EDW = 16
WTILES = 10
WROWS = N // WTILES
TPC = T // 2


def _edges_sc_body(hx_hbm, ed_hbm, src_hbm, dst_hbm, out_hbm,
                   si0, si1, di0, di1, sci0, sci1, g0, g1, e0, e1, s0, s1,
                   accum,
                   sem_i0, sem_i1, sem_g0, sem_g1, sem_e0, sem_e1,
                   sem_s0, sem_s1):
    c = lax.axis_index("c")
    sid = lax.axis_index("s")
    sis = (si0, si1)
    dis = (di0, di1)
    scis = (sci0, sci1)
    gbufs = (g0, g1)
    ebufs = (e0, e1)
    sbufs = (s0, s1)
    sems_i = (sem_i0, sem_i1)
    sems_g = (sem_g0, sem_g1)
    sems_e = (sem_e0, sem_e1)
    sems_s = (sem_s0, sem_s1)
    row0 = sid * WROWS
    my_src = src_hbm.at[sid]
    my_dst = dst_hbm.at[sid]


    def stage_idx(cidx, b):
        pltpu.async_copy(my_src.at[cidx], sis[b], sems_i[b])
        pltpu.async_copy(my_dst.at[cidx], dis[b], sems_i[b])

    def wait_idx(cidx, b):
        pltpu.make_async_copy(my_src.at[cidx], sis[b], sems_i[b]).wait()
        pltpu.make_async_copy(my_dst.at[cidx], dis[b], sems_i[b]).wait()

    def t_body(tt, carry):
        t = c * TPC + tt
        hx_t = hx_hbm.at[t]
        ed_t = ed_hbm.at[t]

        @pl.when(sid < WTILES)
        def _():
            zv = jnp.zeros((16,), jnp.float32)

            @plsc.parallel_loop(0, K, 1, unroll=4)
            def _(i):
                for v in range(FPA // 16):
                    s0[i, pl.ds(v * 16, 16)] = zv

            for i in range(WROWS // K):
                pltpu.sync_copy(s0, accum.at[pl.ds(row0 + i * K, K)])
            pltpu.sync_copy(s0.at[pl.ds(0, WROWS % K)],
                            accum.at[pl.ds(row0 + (WROWS // K) * K,
                                           WROWS % K)])

        plsc.subcore_barrier()

        stage_idx(0, 0)
        stage_idx(1, 1)
        wait_idx(0, 0)
        pltpu.async_copy(hx_t.at[si0], g0, sem_g0)
        pltpu.async_copy(ed_t.at[di0], e0, sem_e0)

        def pair_body(p, carry2):
            for b in range(2):
                cidx = p * 2 + b
                nb = 1 - b
                si_b, di_b, sci_b = sis[b], dis[b], scis[b]
                gb, eb, sb = gbufs[b], ebufs[b], sbufs[b]

                @pl.when(cidx + 1 < NCHUNK)
                def _():
                    wait_idx(cidx + 1, nb)
                    pltpu.async_copy(hx_t.at[sis[nb]], gbufs[nb],
                                     sems_g[nb])
                    pltpu.async_copy(ed_t.at[dis[nb]], ebufs[nb],
                                     sems_e[nb])

                @pl.when(cidx >= 2)
                def _():
                    pltpu.make_async_copy(sb, accum.at[sci_b],
                                          sems_s[b]).wait()

                pltpu.make_async_copy(hx_t.at[si_b], gb, sems_g[b]).wait()
                pltpu.make_async_copy(ed_t.at[di_b], eb, sems_e[b]).wait()

                for k in range(K // 16):
                    sl = pl.ds(k * 16, 16)
                    rows = lax.iota(jnp.int32, 16) + (k * 16)
                    dv = di_b[sl]
                    sci_b[sl] = dv
                    es_v = plsc.load_gather(gb, [rows,
                                                 jnp.full((16,), F,
                                                          jnp.int32)])
                    ed_v = plsc.load_gather(eb, [rows,
                                                 jnp.zeros((16,),
                                                           jnp.int32)])
                    e = es_v + ed_v
                    e = jnp.where(e > 0, e, e * 0.2)
                    w = jnp.exp(e - SHIFT)
                    gid = sid * EPTP + cidx * K + rows
                    w = jnp.where(gid < E, w, 0.0)
                    plsc.store_scatter(sb, [rows,
                                            jnp.full((16,), F, jnp.int32)],
                                       w)

                @plsc.parallel_loop(0, K, 1, unroll=8)
                def _(j):
                    wv = jnp.full((16,), sb[j, pl.ds(FPA - 16, 16)][8])
                    for v in range(F // 16):
                        sl = pl.ds(v * 16, 16)
                        sb[j, sl] = gb[j, sl] * wv

                @pl.when(cidx + 2 < NCHUNK)
                def _():
                    stage_idx(cidx + 2, b)

                pltpu.async_copy(sb, accum.at[sci_b], sems_s[b], add=True)
            return carry2

        lax.fori_loop(0, NCHUNK // 2, pair_body, 0)
        for b in range(2):
            pltpu.make_async_copy(sbufs[b], accum.at[scis[b]],
                                  sems_s[b]).wait()
        plsc.subcore_barrier()

        @pl.when(sid < WTILES)
        def _():
            pltpu.sync_copy(accum.at[pl.ds(row0, WROWS)],
                            out_hbm.at[t].at[pl.ds(row0, WROWS)])

        return carry

    lax.fori_loop(0, TPC, t_body, 0)


def _edges(hx, ed, src, dst):
    mesh = plsc.VectorSubcoreMesh(core_axis_name="c", subcore_axis_name="s")
    ed16 = jnp.broadcast_to(ed[:, :, None], (T, N, EDW))
    pad = jnp.zeros((EP - E,), jnp.int32)
    srcp = jnp.concatenate([src, pad]).reshape(NTILE, NCHUNK, K)
    dstp = jnp.concatenate([dst, pad]).reshape(NTILE, NCHUNK, K)
    out144 = pl.kernel(
        _edges_sc_body,
        out_type=jax.ShapeDtypeStruct((T, N, FPA), jnp.float32),
        mesh=mesh,
        compiler_params=pltpu.CompilerParams(needs_layout_passes=False,
                                             use_tc_tiling_on_sc=False),
        scratch_types=[
            pltpu.VMEM((K,), jnp.int32),
            pltpu.VMEM((K,), jnp.int32),
            pltpu.VMEM((K,), jnp.int32),
            pltpu.VMEM((K,), jnp.int32),
            pltpu.VMEM((K,), jnp.int32),
            pltpu.VMEM((K,), jnp.int32),
            pltpu.VMEM((K, FP), jnp.float32),
            pltpu.VMEM((K, FP), jnp.float32),
            pltpu.VMEM((K, EDW), jnp.float32),
            pltpu.VMEM((K, EDW), jnp.float32),
            pltpu.VMEM((K, FPA), jnp.float32),
            pltpu.VMEM((K, FPA), jnp.float32),
            pltpu.VMEM_SHARED((N, FPA), jnp.float32),
            pltpu.SemaphoreType.DMA,
            pltpu.SemaphoreType.DMA,
            pltpu.SemaphoreType.DMA,
            pltpu.SemaphoreType.DMA,
            pltpu.SemaphoreType.DMA,
            pltpu.SemaphoreType.DMA,
            pltpu.SemaphoreType.DMA,
            pltpu.SemaphoreType.DMA,
        ],
    )(hx, ed16, srcp, dstp)
    return out144


def _post_body(agg_ref, pos_ref, wq_ref, wk_ref, wv_ref, out_ref):
    blk = agg_ref[...]
    den = blk[:, :, F:F + 1]
    agg = blk[:, :, :F]
    hs = agg / (den + 1e-30)
    hs = jnp.where(hs > 0, hs, jnp.exp(jnp.minimum(hs, 0.0)) - 1.0)
    z = hs + pos_ref[...][:, None, :]
    zf = z.reshape(T * NB, F)
    q = jnp.dot(zf, wq_ref[...], preferred_element_type=jnp.float32)
    k = jnp.dot(zf, wk_ref[...], preferred_element_type=jnp.float32)
    v = jnp.dot(zf, wv_ref[...], preferred_element_type=jnp.float32)
    q = q.reshape(T, NB, F) * (1.0 / (F ** 0.5))
    k = k.reshape(T, NB, F)
    v = v.reshape(T, NB, F)
    for t in range(T):
        ss = [jnp.sum(q[t] * k[s], axis=-1) for s in range(t + 1)]
        m = ss[0]
        for s in range(1, t + 1):
            m = jnp.maximum(m, ss[s])
        ws = [jnp.exp(s_ - m) for s_ in ss]
        dsum = ws[0]
        for s in range(1, t + 1):
            dsum = dsum + ws[s]
        acc = ws[0][:, None] * v[0]
        for s in range(1, t + 1):
            acc = acc + ws[s][:, None] * v[s]
        out_ref[:, t, :] = acc / dsum[:, None]


def _post(out144, pos_emb, Wq, Wk, Wv):
    grid = (N // NB,)
    return pl.pallas_call(
        _post_body,
        grid=grid,
        in_specs=[
            pl.BlockSpec((T, NB, FPA), lambda i: (0, i, 0)),
            pl.BlockSpec((T, F), lambda i: (0, 0)),
            pl.BlockSpec((F, F), lambda i: (0, 0)),
            pl.BlockSpec((F, F), lambda i: (0, 0)),
            pl.BlockSpec((F, F), lambda i: (0, 0)),
        ],
        out_specs=pl.BlockSpec((NB, T, F), lambda i: (i, 0, 0)),
        out_shape=jax.ShapeDtypeStruct((N, T, F), jnp.float32),
    )(out144, pos_emb, Wq, Wk, Wv)


def kernel(x, edge_index, W, a_src, a_dst, pos_emb, Wq, Wk, Wv):
    src = edge_index[0]
    dst = edge_index[1]
    hx, ed = _pre(x, W, a_src, a_dst)
    out144 = _edges(hx, ed, src, dst)
    return _post(out144, pos_emb, Wq, Wk, Wv)


# speedup vs baseline: 50.5035x; 1.0012x over previous
import functools

import jax
import jax.numpy as jnp
from jax import lax
from jax.experimental import pallas as pl
from jax.experimental.pallas import tpu as pltpu
from jax.experimental.pallas import tpu_sc as plsc

N = 10000
T = 8
F = 128
E = 320000

SHIFT = 12.0
NB = 1000


def _pre_body(x_ref, w_ref, a2_ref, hx_ref, esd_ref):
    for t in range(T):
        xb = x_ref[:, t, :]
        h = jnp.dot(xb, w_ref[...], preferred_element_type=jnp.float32)
        esd = jnp.dot(h, a2_ref[...], preferred_element_type=jnp.float32)
        hx_ref[t, :, :128] = h
        hx_ref[t, :, 128:] = jnp.broadcast_to(esd[:, 0:1], (NB, 16))
        esd_ref[t, 0, :, 0] = esd[:, 0]
        esd_ref[t, 1, :, 0] = esd[:, 1]


def _pre(x, W, a_src, a_dst):
    a2 = jnp.stack([a_src, a_dst], axis=1)
    grid = (N // NB,)
    hx, esd = pl.pallas_call(
        _pre_body,
        grid=grid,
        in_specs=[
            pl.BlockSpec((NB, T, F), lambda i: (i, 0, 0)),
            pl.BlockSpec((F, F), lambda i: (0, 0)),
            pl.BlockSpec((F, 2), lambda i: (0, 0)),
        ],
        out_specs=[
            pl.BlockSpec((T, NB, 144), lambda i: (0, i, 0)),
            pl.BlockSpec((T, 2, NB, 1), lambda i: (0, 0, i, 0)),
        ],
        out_shape=[
            jax.ShapeDtypeStruct((T, N, 144), jnp.float32),
            jax.ShapeDtypeStruct((T, 2, N, 1), jnp.float32),
        ],
    )(x, W, a2)
    return hx, esd[:, 1, :, 0]


NTILE = 16
K = 64
NCHUNK = 314
EPTP = K * NCHUNK
EP = NTILE * EPTP
FP = 144
FPA = 136
EDW = 16
WTILES = 10
WROWS = N // WTILES
TPC = T // 2


def _edges_sc_body(hx_hbm, ed_hbm, src_hbm, dst_hbm, out_hbm,
                   si0, si1, di0, di1, sci0, sci1, g0, g1, e0, e1, s0, s1,
                   accum,
                   sem_i0, sem_i1, sem_g0, sem_g1, sem_e0, sem_e1,
                   sem_s0, sem_s1):
    c = lax.axis_index("c")
    sid = lax.axis_index("s")
    sis = (si0, si1)
    dis = (di0, di1)
    scis = (sci0, sci1)
    gbufs = (g0, g1)
    ebufs = (e0, e1)
    sbufs = (s0, s1)
    sems_i = (sem_i0, sem_i1)
    sems_g = (sem_g0, sem_g1)
    sems_e = (sem_e0, sem_e1)
    sems_s = (sem_s0, sem_s1)
    row0 = sid * WROWS
    my_src = src_hbm.at[sid]
    my_dst = dst_hbm.at[sid]


    def stage_idx(cidx, b):
        pltpu.async_copy(my_src.at[cidx], sis[b], sems_i[b])
        pltpu.async_copy(my_dst.at[cidx], dis[b], sems_i[b])

    def wait_idx(cidx, b):
        pltpu.make_async_copy(my_src.at[cidx], sis[b], sems_i[b]).wait()
        pltpu.make_async_copy(my_dst.at[cidx], dis[b], sems_i[b]).wait()

    def t_body(tt, carry):
        t = c * TPC + tt
        hx_t = hx_hbm.at[t]
        ed_t = ed_hbm.at[t]

        @pl.when(sid < WTILES)
        def _():
            zv = jnp.zeros((16,), jnp.float32)

            @plsc.parallel_loop(0, K, 1, unroll=4)
            def _(i):
                for v0 in list(range(0, F, 16)) + [FPA - 16]:
                    s0[i, pl.ds(v0, 16)] = zv

            for i in range(WROWS // K):
                pltpu.sync_copy(s0, accum.at[pl.ds(row0 + i * K, K)])
            pltpu.sync_copy(s0.at[pl.ds(0, WROWS % K)],
                            accum.at[pl.ds(row0 + (WROWS // K) * K,
                                           WROWS % K)])

        plsc.subcore_barrier()

        stage_idx(0, 0)
        stage_idx(1, 1)
        wait_idx(0, 0)
        pltpu.async_copy(hx_t.at[si0], g0, sem_g0)
        pltpu.async_copy(ed_t.at[di0], e0, sem_e0)

        def pair_body(p, carry2):
            for b in range(2):
                cidx = p * 2 + b
                nb = 1 - b
                si_b, di_b, sci_b = sis[b], dis[b], scis[b]
                gb, eb, sb = gbufs[b], ebufs[b], sbufs[b]

                @pl.when(cidx + 1 < NCHUNK)
                def _():
                    wait_idx(cidx + 1, nb)
                    pltpu.async_copy(hx_t.at[sis[nb]], gbufs[nb],
                                     sems_g[nb])
                    pltpu.async_copy(ed_t.at[dis[nb]], ebufs[nb],
                                     sems_e[nb])

                @pl.when(cidx >= 2)
                def _():
                    pltpu.make_async_copy(sb, accum.at[sci_b],
                                          sems_s[b]).wait()

                pltpu.make_async_copy(hx_t.at[si_b], gb, sems_g[b]).wait()
                pltpu.make_async_copy(ed_t.at[di_b], eb, sems_e[b]).wait()

                for k in range(K // 16):
                    sl = pl.ds(k * 16, 16)
                    rows = lax.iota(jnp.int32, 16) + (k * 16)
                    dv = di_b[sl]
                    sci_b[sl] = dv
                    es_v = plsc.load_gather(gb, [rows,
                                                 jnp.full((16,), F,
                                                          jnp.int32)])
                    ed_v = plsc.load_gather(eb, [rows,
                                                 jnp.zeros((16,),
                                                           jnp.int32)])
                    e = es_v + ed_v
                    e = jnp.where(e > 0, e, e * 0.2)
                    w = jnp.exp(e - SHIFT)
                    gid = sid * EPTP + cidx * K + rows
                    w = jnp.where(gid < E, w, 0.0)
                    plsc.store_scatter(sb, [rows,
                                            jnp.full((16,), F, jnp.int32)],
                                       w)

                @plsc.parallel_loop(0, K, 1, unroll=8)
                def _(j):
                    wv = jnp.full((16,), sb[j, pl.ds(FPA - 16, 16)][8])
                    for v in range(F // 16):
                        sl = pl.ds(v * 16, 16)
                        sb[j, sl] = gb[j, sl] * wv

                @pl.when(cidx + 2 < NCHUNK)
                def _():
                    stage_idx(cidx + 2, b)

                pltpu.async_copy(sb, accum.at[sci_b], sems_s[b], add=True)
            return carry2

        lax.fori_loop(0, NCHUNK // 2, pair_body, 0)
        for b in range(2):
            pltpu.make_async_copy(sbufs[b], accum.at[scis[b]],
                                  sems_s[b]).wait()
        plsc.subcore_barrier()

        @pl.when(sid < WTILES)
        def _():
            pltpu.sync_copy(accum.at[pl.ds(row0, WROWS)],
                            out_hbm.at[t].at[pl.ds(row0, WROWS)])

        return carry

    lax.fori_loop(0, TPC, t_body, 0)


def _edges(hx, ed, src, dst):
    mesh = plsc.VectorSubcoreMesh(core_axis_name="c", subcore_axis_name="s")
    ed16 = jnp.broadcast_to(ed[:, :, None], (T, N, EDW))
    pad = jnp.zeros((EP - E,), jnp.int32)
    srcp = jnp.concatenate([src, pad]).reshape(NTILE, NCHUNK, K)
    dstp = jnp.concatenate([dst, pad]).reshape(NTILE, NCHUNK, K)
    out144 = pl.kernel(
        _edges_sc_body,
        out_type=jax.ShapeDtypeStruct((T, N, FPA), jnp.float32),
        mesh=mesh,
        compiler_params=pltpu.CompilerParams(needs_layout_passes=False,
                                             use_tc_tiling_on_sc=False),
        scratch_types=[
            pltpu.VMEM((K,), jnp.int32),
            pltpu.VMEM((K,), jnp.int32),
            pltpu.VMEM((K,), jnp.int32),
            pltpu.VMEM((K,), jnp.int32),
            pltpu.VMEM((K,), jnp.int32),
            pltpu.VMEM((K,), jnp.int32),
            pltpu.VMEM((K, FP), jnp.float32),
            pltpu.VMEM((K, FP), jnp.float32),
            pltpu.VMEM((K, EDW), jnp.float32),
            pltpu.VMEM((K, EDW), jnp.float32),
            pltpu.VMEM((K, FPA), jnp.float32),
            pltpu.VMEM((K, FPA), jnp.float32),
            pltpu.VMEM_SHARED((N, FPA), jnp.float32),
            pltpu.SemaphoreType.DMA,
            pltpu.SemaphoreType.DMA,
            pltpu.SemaphoreType.DMA,
            pltpu.SemaphoreType.DMA,
            pltpu.SemaphoreType.DMA,
            pltpu.SemaphoreType.DMA,
            pltpu.SemaphoreType.DMA,
            pltpu.SemaphoreType.DMA,
        ],
    )(hx, ed16, srcp, dstp)
    return out144


def _post_body(agg_ref, pos_ref, wq_ref, wk_ref, wv_ref, out_ref):
    blk = agg_ref[...]
    den = blk[:, :, F:F + 1]
    agg = blk[:, :, :F]
    hs = agg / (den + 1e-30)
    hs = jnp.where(hs > 0, hs, jnp.exp(jnp.minimum(hs, 0.0)) - 1.0)
    z = hs + pos_ref[...][:, None, :]
    zf = z.reshape(T * NB, F)
    q = jnp.dot(zf, wq_ref[...], preferred_element_type=jnp.float32)
    k = jnp.dot(zf, wk_ref[...], preferred_element_type=jnp.float32)
    v = jnp.dot(zf, wv_ref[...], preferred_element_type=jnp.float32)
    q = q.reshape(T, NB, F) * (1.0 / (F ** 0.5))
    k = k.reshape(T, NB, F)
    v = v.reshape(T, NB, F)
    for t in range(T):
        ss = [jnp.sum(q[t] * k[s], axis=-1) for s in range(t + 1)]
        m = ss[0]
        for s in range(1, t + 1):
            m = jnp.maximum(m, ss[s])
        ws = [jnp.exp(s_ - m) for s_ in ss]
        dsum = ws[0]
        for s in range(1, t + 1):
            dsum = dsum + ws[s]
        acc = ws[0][:, None] * v[0]
        for s in range(1, t + 1):
            acc = acc + ws[s][:, None] * v[s]
        out_ref[:, t, :] = acc / dsum[:, None]


def _post(out144, pos_emb, Wq, Wk, Wv):
    grid = (N // NB,)
    return pl.pallas_call(
        _post_body,
        grid=grid,
        in_specs=[
            pl.BlockSpec((T, NB, FPA), lambda i: (0, i, 0)),
            pl.BlockSpec((T, F), lambda i: (0, 0)),
            pl.BlockSpec((F, F), lambda i: (0, 0)),
            pl.BlockSpec((F, F), lambda i: (0, 0)),
            pl.BlockSpec((F, F), lambda i: (0, 0)),
        ],
        out_specs=pl.BlockSpec((NB, T, F), lambda i: (i, 0, 0)),
        out_shape=jax.ShapeDtypeStruct((N, T, F), jnp.float32),
    )(out144, pos_emb, Wq, Wk, Wv)


def kernel(x, edge_index, W, a_src, a_dst, pos_emb, Wq, Wk, Wv):
    src = edge_index[0]
    dst = edge_index[1]
    hx, ed = _pre(x, W, a_src, a_dst)
    out144 = _edges(hx, ed, src, dst)
    return _post(out144, pos_emb, Wq, Wk, Wv)
